# Initial kernel scaffold; baseline (speedup 1.0000x reference)
import jax
import jax.numpy as jnp
from jax.experimental import pallas as pl


def kernel(x, edge_index, W_gcn, b_gcn, W_proj, b_proj):
    raise NotImplementedError("write your pallas kernel here")


# speedup vs baseline: 386.1176x; 386.1176x over previous
import functools

import jax
import jax.numpy as jnp
from jax import lax
from jax.experimental import pallas as pl
from jax.experimental.pallas import tpu as pltpu
from jax.experimental.pallas import tpu_sc as plsc

N = 100000
E = 3200000
HID = 128
HOR = 12
B = 4
T = 12

NC = 2
NS = 16
NW = NC * NS
LGRP = 128
N_PAD = 100352
SL = N_PAD // NS
E_ROWS = 25088
ROWS_PT = E_ROWS // NW
SR = 16
N_CH = ROWS_PT // SR
BLK = 2048
GRID_N = N_PAD // BLK

_sc_mesh = plsc.VectorSubcoreMesh(core_axis_name="c", subcore_axis_name="s")


@functools.partial(
    pl.kernel,
    out_type=(
        jax.ShapeDtypeStruct((NC, N_PAD), jnp.float32),
        jax.ShapeDtypeStruct((NC, N_PAD), jnp.float32),
    ),
    mesh=_sc_mesh,
    scratch_types=(
        pltpu.VMEM_SHARED((N_PAD,), jnp.float32),
        pltpu.VMEM_SHARED((N_PAD,), jnp.float32),
        pltpu.VMEM((SR, LGRP), jnp.int32),
        pltpu.VMEM((LGRP,), jnp.float32),
    ),
)
def _degrees(rows_h, cols_h, zeros_h, degs_h, degc_h, hs, hc, idx_st, ones_v):
    c = lax.axis_index("c")
    s = lax.axis_index("s")
    for k in range(LGRP // 16):
        ones_v[pl.ds(k * 16, 16)] = jnp.ones((16,), jnp.float32)
    pltpu.sync_copy(zeros_h.at[pl.ds(s * SL, SL)], hs.at[pl.ds(s * SL, SL)])
    pltpu.sync_copy(zeros_h.at[pl.ds(s * SL, SL)], hc.at[pl.ds(s * SL, SL)])
    plsc.subcore_barrier()

    base = (c * NS + s) * ROWS_PT

    @pl.loop(0, N_CH)
    def _row_chunk(i):
        pltpu.sync_copy(rows_h.at[pl.ds(base + i * SR, SR)], idx_st)

        @pl.loop(0, SR)
        def _row_grp(j):
            pltpu.sync_copy(ones_v, hs.at[idx_st.at[j]], add=True)

    @pl.loop(0, N_CH)
    def _col_chunk(i):
        pltpu.sync_copy(cols_h.at[pl.ds(base + i * SR, SR)], idx_st)

        @pl.loop(0, SR)
        def _col_grp(j):
            pltpu.sync_copy(ones_v, hc.at[idx_st.at[j]], add=True)

    plsc.subcore_barrier()
    pltpu.sync_copy(hs.at[pl.ds(s * SL, SL)], degs_h.at[c, pl.ds(s * SL, SL)])
    pltpu.sync_copy(hc.at[pl.ds(s * SL, SL)], degc_h.at[c, pl.ds(s * SL, SL)])


def _coef_body(x_r, ds_r, dc_r, gt_r, dinv_r):
    i = pl.program_id(0)
    m4 = jnp.sum(x_r[...], axis=1) * (1.0 / T)
    dsrc = ds_r[0] + ds_r[1]
    deg = dc_r[0] + dc_r[1] + 1.0
    dinv = lax.rsqrt(deg)
    iso = 1.0 / (dsrc + 0.001)
    coef = dinv * iso
    nidx = lax.broadcasted_iota(jnp.int32, (BLK,), 0) + i * BLK
    valid = nidx < N
    gt_r[...] = jnp.where(valid[None, :], m4 * coef[None, :], 0.0)
    dinv_r[...] = jnp.where(valid, dinv, 0.0).reshape(1, BLK)


_coef = pl.pallas_call(
    _coef_body,
    grid=(GRID_N,),
    in_specs=[
        pl.BlockSpec((B, T, BLK), lambda i: (0, 0, i)),
        pl.BlockSpec((2, BLK), lambda i: (0, i)),
        pl.BlockSpec((2, BLK), lambda i: (0, i)),
    ],
    out_specs=[
        pl.BlockSpec((B, BLK), lambda i: (0, i)),
        pl.BlockSpec((1, BLK), lambda i: (0, i)),
    ],
    out_shape=[
        jax.ShapeDtypeStruct((B, N_PAD), jnp.float32),
        jax.ShapeDtypeStruct((1, N_PAD), jnp.float32),
    ],
)


@functools.partial(
    pl.kernel,
    out_type=jax.ShapeDtypeStruct((NC, B, N_PAD), jnp.float32),
    mesh=_sc_mesh,
    scratch_types=(
        pltpu.VMEM_SHARED((N_PAD,), jnp.float32),
        pltpu.VMEM_SHARED((N_PAD,), jnp.float32),
        pltpu.VMEM_SHARED((N_PAD,), jnp.float32),
        pltpu.VMEM_SHARED((N_PAD,), jnp.float32),
        pltpu.VMEM_SHARED((N_PAD,), jnp.float32),
        pltpu.VMEM_SHARED((N_PAD,), jnp.float32),
        pltpu.VMEM_SHARED((N_PAD,), jnp.float32),
        pltpu.VMEM_SHARED((N_PAD,), jnp.float32),
        pltpu.VMEM((SR, LGRP), jnp.int32),
        pltpu.VMEM((SR, LGRP), jnp.int32),
        pltpu.VMEM((LGRP,), jnp.float32),
        pltpu.VMEM((LGRP,), jnp.float32),
        pltpu.VMEM((LGRP,), jnp.float32),
        pltpu.VMEM((LGRP,), jnp.float32),
        pltpu.SemaphoreType.DMA,
    ),
)
def _edge_pass(
    rows_h, cols_h, gt_h, zeros_h, y_h,
    g0, g1, g2, g3, y0, y1, y2, y3, ridx, cidx, v0, v1, v2, v3, sem,
):
    c = lax.axis_index("c")
    s = lax.axis_index("s")
    gs = (g0, g1, g2, g3)
    ys = (y0, y1, y2, y3)
    vs = (v0, v1, v2, v3)
    for b in range(B):
        pltpu.sync_copy(gt_h.at[b, pl.ds(s * SL, SL)], gs[b].at[pl.ds(s * SL, SL)])
        pltpu.sync_copy(zeros_h.at[pl.ds(s * SL, SL)], ys[b].at[pl.ds(s * SL, SL)])
    plsc.subcore_barrier()

    base = (c * NS + s) * ROWS_PT

    @pl.loop(0, N_CH)
    def _chunk(i):
        pltpu.sync_copy(rows_h.at[pl.ds(base + i * SR, SR)], ridx)
        pltpu.sync_copy(cols_h.at[pl.ds(base + i * SR, SR)], cidx)

        @pl.loop(0, SR)
        def _grp(j):
            gathers = [
                pltpu.async_copy(gs[b].at[ridx.at[j]], vs[b], sem)
                for b in range(B)
            ]
            for d in gathers:
                d.wait()
            scatters = [
                pltpu.async_copy(vs[b], ys[b].at[cidx.at[j]], sem, add=True)
                for b in range(B)
            ]
            for d in scatters:
                d.wait()

    plsc.subcore_barrier()
    for b in range(B):
        pltpu.sync_copy(ys[b].at[pl.ds(s * SL, SL)], y_h.at[c, b, pl.ds(s * SL, SL)])


def _final_body(y_r, gt_r, dinv_r, wg_r, wp_r, bg_r, bp_r, out_r):
    s4 = (y_r[0] + y_r[1] + gt_r[...]) * dinv_r[...]
    v = jnp.dot(wg_r[...], wp_r[...], preferred_element_type=jnp.float32)
    c0 = jnp.dot(bg_r[...], wp_r[...], preferred_element_type=jnp.float32)
    c0 = c0 + bp_r[...]
    for b in range(B):
        outer = lax.dot_general(
            s4[b : b + 1], v, (((0,), (0,)), ((), ())),
            preferred_element_type=jnp.float32,
        )
        out_r[b] = outer + c0


_final = pl.pallas_call(
    _final_body,
    grid=(GRID_N,),
    in_specs=[
        pl.BlockSpec((NC, B, BLK), lambda i: (0, 0, i)),
        pl.BlockSpec((B, BLK), lambda i: (0, i)),
        pl.BlockSpec((1, BLK), lambda i: (0, i)),
        pl.BlockSpec((1, HID), lambda i: (0, 0)),
        pl.BlockSpec((HID, HOR), lambda i: (0, 0)),
        pl.BlockSpec((1, HID), lambda i: (0, 0)),
        pl.BlockSpec((1, HOR), lambda i: (0, 0)),
    ],
    out_specs=pl.BlockSpec((B, BLK, HOR), lambda i: (0, i, 0)),
    out_shape=jax.ShapeDtypeStruct((B, N, HOR), jnp.float32),
)


def kernel(x, edge_index, W_gcn, b_gcn, W_proj, b_proj):
    ei = edge_index.astype(jnp.int32)
    pad = jnp.full((E_ROWS * LGRP - E,), N_PAD - 1, jnp.int32)
    rows2d = jnp.concatenate([ei[0], pad]).reshape(E_ROWS, LGRP)
    cols2d = jnp.concatenate([ei[1], pad]).reshape(E_ROWS, LGRP)
    zeros1 = jnp.zeros((N_PAD,), jnp.float32)
    x3 = x[..., 0]

    degs_p, degc_p = _degrees(rows2d, cols2d, zeros1)
    gt4, dinv1 = _coef(x3, degs_p, degc_p)
    y_p = _edge_pass(rows2d, cols2d, gt4, zeros1)
    return _final(
        y_p, gt4, dinv1,
        W_gcn, W_proj,
        b_gcn.reshape(1, HID), b_proj.reshape(1, HOR),
    )


# speedup vs baseline: 469.3285x; 1.2155x over previous
import functools

import jax
import jax.numpy as jnp
from jax import lax
from jax.experimental import pallas as pl
from jax.experimental.pallas import tpu as pltpu
from jax.experimental.pallas import tpu_sc as plsc

N = 100000
E = 3200000
HID = 128
HOR = 12
B = 4
T = 12

NC = 2
NS = 16
NW = NC * NS
LGRP = 128
N_PAD = 100352
SL = N_PAD // NS
E_ROWS = 25088
ROWS_PT = E_ROWS // NW
SR = 16
N_CH = ROWS_PT // SR
BLK = 2048
GRID_N = N_PAD // BLK

_sc_mesh = plsc.VectorSubcoreMesh(core_axis_name="c", subcore_axis_name="s")


@functools.partial(
    pl.kernel,
    out_type=(
        jax.ShapeDtypeStruct((NC, N_PAD), jnp.float32),
        jax.ShapeDtypeStruct((NC, N_PAD), jnp.float32),
    ),
    mesh=_sc_mesh,
    scratch_types=(
        pltpu.VMEM_SHARED((N_PAD,), jnp.float32),
        pltpu.VMEM_SHARED((N_PAD,), jnp.float32),
        pltpu.VMEM((SR, LGRP), jnp.int32),
        pltpu.VMEM((SR, LGRP), jnp.int32),
        pltpu.VMEM((LGRP,), jnp.float32),
        pltpu.SemaphoreType.DMA,
    ),
)
def _degrees(rows_h, cols_h, zeros_h, degs_h, degc_h, hs, hc, ridx, cidx, ones_v, sem):
    c = lax.axis_index("c")
    s = lax.axis_index("s")
    for k in range(LGRP // 16):
        ones_v[pl.ds(k * 16, 16)] = jnp.ones((16,), jnp.float32)
    pltpu.sync_copy(zeros_h.at[pl.ds(s * SL, SL)], hs.at[pl.ds(s * SL, SL)])
    pltpu.sync_copy(zeros_h.at[pl.ds(s * SL, SL)], hc.at[pl.ds(s * SL, SL)])
    plsc.subcore_barrier()

    base = (c * NS + s) * ROWS_PT

    @pl.loop(0, N_CH)
    def _chunk(i):
        pltpu.sync_copy(rows_h.at[pl.ds(base + i * SR, SR)], ridx)
        pltpu.sync_copy(cols_h.at[pl.ds(base + i * SR, SR)], cidx)

        @pl.loop(0, SR)
        def _grp(j):
            d1 = pltpu.async_copy(ones_v, hs.at[ridx.at[j]], sem, add=True)
            d2 = pltpu.async_copy(ones_v, hc.at[cidx.at[j]], sem, add=True)
            d1.wait()
            d2.wait()

    plsc.subcore_barrier()
    pltpu.sync_copy(hs.at[pl.ds(s * SL, SL)], degs_h.at[c, pl.ds(s * SL, SL)])
    pltpu.sync_copy(hc.at[pl.ds(s * SL, SL)], degc_h.at[c, pl.ds(s * SL, SL)])


def _coef_body(x_r, ds_r, dc_r, gt_r, dinv_r):
    i = pl.program_id(0)
    m4 = jnp.sum(x_r[...], axis=1) * (1.0 / T)
    dsrc = ds_r[0] + ds_r[1]
    deg = dc_r[0] + dc_r[1] + 1.0
    dinv = lax.rsqrt(deg)
    iso = 1.0 / (dsrc + 0.001)
    coef = dinv * iso
    nidx = lax.broadcasted_iota(jnp.int32, (BLK,), 0) + i * BLK
    valid = nidx < N
    gt_r[...] = jnp.where(valid[None, :], m4 * coef[None, :], 0.0)
    dinv_r[...] = jnp.where(valid, dinv, 0.0).reshape(1, BLK)


_coef = pl.pallas_call(
    _coef_body,
    grid=(GRID_N,),
    in_specs=[
        pl.BlockSpec((B, T, BLK), lambda i: (0, 0, i)),
        pl.BlockSpec((2, BLK), lambda i: (0, i)),
        pl.BlockSpec((2, BLK), lambda i: (0, i)),
    ],
    out_specs=[
        pl.BlockSpec((B, BLK), lambda i: (0, i)),
        pl.BlockSpec((1, BLK), lambda i: (0, i)),
    ],
    out_shape=[
        jax.ShapeDtypeStruct((B, N_PAD), jnp.float32),
        jax.ShapeDtypeStruct((1, N_PAD), jnp.float32),
    ],
)


@functools.partial(
    pl.kernel,
    out_type=jax.ShapeDtypeStruct((NC, B, N_PAD), jnp.float32),
    mesh=_sc_mesh,
    scratch_types=(
        pltpu.VMEM_SHARED((N_PAD,), jnp.float32),
        pltpu.VMEM_SHARED((N_PAD,), jnp.float32),
        pltpu.VMEM_SHARED((N_PAD,), jnp.float32),
        pltpu.VMEM_SHARED((N_PAD,), jnp.float32),
        pltpu.VMEM_SHARED((N_PAD,), jnp.float32),
        pltpu.VMEM_SHARED((N_PAD,), jnp.float32),
        pltpu.VMEM_SHARED((N_PAD,), jnp.float32),
        pltpu.VMEM_SHARED((N_PAD,), jnp.float32),
        pltpu.VMEM((SR, LGRP), jnp.int32),
        pltpu.VMEM((SR, LGRP), jnp.int32),
        pltpu.VMEM((2, B, LGRP), jnp.float32),
        pltpu.SemaphoreType.DMA,
        pltpu.SemaphoreType.DMA,
    ),
)
def _edge_pass(
    rows_h, cols_h, gt_h, zeros_h, y_h,
    g0, g1, g2, g3, y0, y1, y2, y3, ridx, cidx, vbuf, semg, sems,
):
    c = lax.axis_index("c")
    s = lax.axis_index("s")
    gs = (g0, g1, g2, g3)
    ys = (y0, y1, y2, y3)
    for b in range(B):
        pltpu.sync_copy(gt_h.at[b, pl.ds(s * SL, SL)], gs[b].at[pl.ds(s * SL, SL)])
        pltpu.sync_copy(zeros_h.at[pl.ds(s * SL, SL)], ys[b].at[pl.ds(s * SL, SL)])
    plsc.subcore_barrier()

    base = (c * NS + s) * ROWS_PT

    def fire_g(p, j):
        for b in range(B):
            pltpu.async_copy(gs[b].at[ridx.at[j]], vbuf.at[p, b], semg)

    def wait_g(p):
        for b in range(B):
            pltpu.make_async_copy(gs[b].at[ridx.at[0]], vbuf.at[p, b], semg).wait()

    def fire_s(p, j):
        for b in range(B):
            pltpu.async_copy(vbuf.at[p, b], ys[b].at[cidx.at[j]], sems, add=True)

    def wait_s(p):
        for b in range(B):
            pltpu.make_async_copy(vbuf.at[p, b], ys[b].at[cidx.at[0]], sems).wait()

    @pl.loop(0, N_CH)
    def _chunk(i):
        pltpu.sync_copy(rows_h.at[pl.ds(base + i * SR, SR)], ridx)
        pltpu.sync_copy(cols_h.at[pl.ds(base + i * SR, SR)], cidx)
        fire_g(0, 0)

        @pl.loop(0, SR // 2)
        def _pair(h):
            j0 = h * 2
            wait_g(0)
            fire_s(0, j0)
            fire_g(1, j0 + 1)
            wait_s(0)
            wait_g(1)
            fire_s(1, j0 + 1)

            @pl.when(h < SR // 2 - 1)
            def _prefetch():
                fire_g(0, j0 + 2)

            wait_s(1)

    plsc.subcore_barrier()
    for b in range(B):
        pltpu.sync_copy(ys[b].at[pl.ds(s * SL, SL)], y_h.at[c, b, pl.ds(s * SL, SL)])


def _final_body(y_r, gt_r, dinv_r, wg_r, wp_r, bg_r, bp_r, out_r):
    s4 = (y_r[0] + y_r[1] + gt_r[...]) * dinv_r[...]
    v = jnp.dot(wg_r[...], wp_r[...], preferred_element_type=jnp.float32)
    c0 = jnp.dot(bg_r[...], wp_r[...], preferred_element_type=jnp.float32)
    c0 = c0 + bp_r[...]
    for b in range(B):
        outer = lax.dot_general(
            s4[b : b + 1], v, (((0,), (0,)), ((), ())),
            preferred_element_type=jnp.float32,
        )
        out_r[b] = outer + c0


_final = pl.pallas_call(
    _final_body,
    grid=(GRID_N,),
    in_specs=[
        pl.BlockSpec((NC, B, BLK), lambda i: (0, 0, i)),
        pl.BlockSpec((B, BLK), lambda i: (0, i)),
        pl.BlockSpec((1, BLK), lambda i: (0, i)),
        pl.BlockSpec((1, HID), lambda i: (0, 0)),
        pl.BlockSpec((HID, HOR), lambda i: (0, 0)),
        pl.BlockSpec((1, HID), lambda i: (0, 0)),
        pl.BlockSpec((1, HOR), lambda i: (0, 0)),
    ],
    out_specs=pl.BlockSpec((B, BLK, HOR), lambda i: (0, i, 0)),
    out_shape=jax.ShapeDtypeStruct((B, N, HOR), jnp.float32),
)


def kernel(x, edge_index, W_gcn, b_gcn, W_proj, b_proj):
    ei = edge_index.astype(jnp.int32)
    pad = jnp.full((E_ROWS * LGRP - E,), N_PAD - 1, jnp.int32)
    rows2d = jnp.concatenate([ei[0], pad]).reshape(E_ROWS, LGRP)
    cols2d = jnp.concatenate([ei[1], pad]).reshape(E_ROWS, LGRP)
    zeros1 = jnp.zeros((N_PAD,), jnp.float32)
    x3 = x[..., 0]

    degs_p, degc_p = _degrees(rows2d, cols2d, zeros1)
    gt4, dinv1 = _coef(x3, degs_p, degc_p)
    y_p = _edge_pass(rows2d, cols2d, gt4, zeros1)
    return _final(
        y_p, gt4, dinv1,
        W_gcn, W_proj,
        b_gcn.reshape(1, HID), b_proj.reshape(1, HOR),
    )


# speedup vs baseline: 479.3200x; 1.0213x over previous
import functools

import jax
import jax.numpy as jnp
from jax import lax
from jax.experimental import pallas as pl
from jax.experimental.pallas import tpu as pltpu
from jax.experimental.pallas import tpu_sc as plsc

N = 100000
E = 3200000
HID = 128
HOR = 12
B = 4
T = 12

NC = 2
NS = 16
NW = NC * NS
LGRP = 128
N_PAD = 100352
SL = N_PAD // NS
E_ROWS = 25088
ROWS_PT = E_ROWS // NW
SR = 16
N_CH = ROWS_PT // SR
BLK = 2048
GRID_N = N_PAD // BLK

_sc_mesh = plsc.VectorSubcoreMesh(core_axis_name="c", subcore_axis_name="s")


@functools.partial(
    pl.kernel,
    out_type=(
        jax.ShapeDtypeStruct((NC, N_PAD), jnp.float32),
        jax.ShapeDtypeStruct((NC, N_PAD), jnp.float32),
    ),
    mesh=_sc_mesh,
    scratch_types=(
        pltpu.VMEM_SHARED((N_PAD,), jnp.float32),
        pltpu.VMEM_SHARED((N_PAD,), jnp.float32),
        pltpu.VMEM((SR, LGRP), jnp.int32),
        pltpu.VMEM((SR, LGRP), jnp.int32),
        pltpu.VMEM((LGRP,), jnp.float32),
        pltpu.SemaphoreType.DMA,
    ),
)
def _degrees(rows_h, cols_h, zeros_h, degs_h, degc_h, hs, hc, ridx, cidx, ones_v, sem):
    c = lax.axis_index("c")
    s = lax.axis_index("s")
    for k in range(LGRP // 16):
        ones_v[pl.ds(k * 16, 16)] = jnp.ones((16,), jnp.float32)
    pltpu.sync_copy(zeros_h.at[pl.ds(s * SL, SL)], hs.at[pl.ds(s * SL, SL)])
    pltpu.sync_copy(zeros_h.at[pl.ds(s * SL, SL)], hc.at[pl.ds(s * SL, SL)])
    plsc.subcore_barrier()

    base = (c * NS + s) * ROWS_PT

    @pl.loop(0, N_CH)
    def _chunk(i):
        pltpu.sync_copy(rows_h.at[pl.ds(base + i * SR, SR)], ridx)
        pltpu.sync_copy(cols_h.at[pl.ds(base + i * SR, SR)], cidx)

        @pl.loop(0, SR // 2)
        def _grp(h):
            j0 = h * 2
            ds = [
                pltpu.async_copy(ones_v, hs.at[ridx.at[j0]], sem, add=True),
                pltpu.async_copy(ones_v, hc.at[cidx.at[j0]], sem, add=True),
                pltpu.async_copy(ones_v, hs.at[ridx.at[j0 + 1]], sem, add=True),
                pltpu.async_copy(ones_v, hc.at[cidx.at[j0 + 1]], sem, add=True),
            ]
            for d in ds:
                d.wait()

    plsc.subcore_barrier()
    pltpu.sync_copy(hs.at[pl.ds(s * SL, SL)], degs_h.at[c, pl.ds(s * SL, SL)])
    pltpu.sync_copy(hc.at[pl.ds(s * SL, SL)], degc_h.at[c, pl.ds(s * SL, SL)])


def _coef_body(x_r, ds_r, dc_r, gt_r, dinv_r):
    i = pl.program_id(0)
    m4 = jnp.sum(x_r[...], axis=1) * (1.0 / T)
    dsrc = ds_r[0] + ds_r[1]
    deg = dc_r[0] + dc_r[1] + 1.0
    dinv = lax.rsqrt(deg)
    iso = 1.0 / (dsrc + 0.001)
    coef = dinv * iso
    nidx = lax.broadcasted_iota(jnp.int32, (BLK,), 0) + i * BLK
    valid = nidx < N
    gt_r[...] = jnp.where(valid[None, :], m4 * coef[None, :], 0.0)
    dinv_r[...] = jnp.where(valid, dinv, 0.0).reshape(1, BLK)


_coef = pl.pallas_call(
    _coef_body,
    grid=(GRID_N,),
    in_specs=[
        pl.BlockSpec((B, T, BLK), lambda i: (0, 0, i)),
        pl.BlockSpec((2, BLK), lambda i: (0, i)),
        pl.BlockSpec((2, BLK), lambda i: (0, i)),
    ],
    out_specs=[
        pl.BlockSpec((B, BLK), lambda i: (0, i)),
        pl.BlockSpec((1, BLK), lambda i: (0, i)),
    ],
    out_shape=[
        jax.ShapeDtypeStruct((B, N_PAD), jnp.float32),
        jax.ShapeDtypeStruct((1, N_PAD), jnp.float32),
    ],
)


@functools.partial(
    pl.kernel,
    out_type=jax.ShapeDtypeStruct((NC, B, N_PAD), jnp.float32),
    mesh=_sc_mesh,
    scratch_types=(
        pltpu.VMEM_SHARED((N_PAD,), jnp.float32),
        pltpu.VMEM_SHARED((N_PAD,), jnp.float32),
        pltpu.VMEM_SHARED((N_PAD,), jnp.float32),
        pltpu.VMEM_SHARED((N_PAD,), jnp.float32),
        pltpu.VMEM_SHARED((N_PAD,), jnp.float32),
        pltpu.VMEM_SHARED((N_PAD,), jnp.float32),
        pltpu.VMEM_SHARED((N_PAD,), jnp.float32),
        pltpu.VMEM_SHARED((N_PAD,), jnp.float32),
        pltpu.VMEM((SR, LGRP), jnp.int32),
        pltpu.VMEM((SR, LGRP), jnp.int32),
        pltpu.VMEM((2, B, LGRP), jnp.float32),
        pltpu.SemaphoreType.DMA,
        pltpu.SemaphoreType.DMA,
    ),
)
def _edge_pass(
    rows_h, cols_h, gt_h, zeros_h, y_h,
    g0, g1, g2, g3, y0, y1, y2, y3, ridx, cidx, vbuf, semg, sems,
):
    c = lax.axis_index("c")
    s = lax.axis_index("s")
    gs = (g0, g1, g2, g3)
    ys = (y0, y1, y2, y3)
    for b in range(B):
        pltpu.sync_copy(gt_h.at[b, pl.ds(s * SL, SL)], gs[b].at[pl.ds(s * SL, SL)])
        pltpu.sync_copy(zeros_h.at[pl.ds(s * SL, SL)], ys[b].at[pl.ds(s * SL, SL)])
    plsc.subcore_barrier()

    base = (c * NS + s) * ROWS_PT

    def fire_g(p, j):
        for b in range(B):
            pltpu.async_copy(gs[b].at[ridx.at[j]], vbuf.at[p, b], semg)

    def wait_g(p):
        for b in range(B):
            pltpu.make_async_copy(gs[b].at[ridx.at[0]], vbuf.at[p, b], semg).wait()

    def fire_s(p, j):
        for b in range(B):
            pltpu.async_copy(vbuf.at[p, b], ys[b].at[cidx.at[j]], sems, add=True)

    def wait_s(p):
        for b in range(B):
            pltpu.make_async_copy(vbuf.at[p, b], ys[b].at[cidx.at[0]], sems).wait()

    @pl.loop(0, N_CH)
    def _chunk(i):
        pltpu.sync_copy(rows_h.at[pl.ds(base + i * SR, SR)], ridx)
        pltpu.sync_copy(cols_h.at[pl.ds(base + i * SR, SR)], cidx)
        fire_g(0, 0)

        @pl.loop(0, SR // 2)
        def _pair(h):
            j0 = h * 2
            wait_g(0)
            fire_s(0, j0)
            fire_g(1, j0 + 1)
            wait_s(0)
            wait_g(1)
            fire_s(1, j0 + 1)

            @pl.when(h < SR // 2 - 1)
            def _prefetch():
                fire_g(0, j0 + 2)

            wait_s(1)

    plsc.subcore_barrier()
    for b in range(B):
        pltpu.sync_copy(ys[b].at[pl.ds(s * SL, SL)], y_h.at[c, b, pl.ds(s * SL, SL)])


def _final_body(y_r, gt_r, dinv_r, wg_r, wp_r, bg_r, bp_r, out_r):
    s4 = (y_r[0] + y_r[1] + gt_r[...]) * dinv_r[...]
    v = jnp.dot(wg_r[...], wp_r[...], preferred_element_type=jnp.float32)
    c0 = jnp.dot(bg_r[...], wp_r[...], preferred_element_type=jnp.float32)
    c0 = c0 + bp_r[...]
    for b in range(B):
        outer = lax.dot_general(
            s4[b : b + 1], v, (((0,), (0,)), ((), ())),
            preferred_element_type=jnp.float32,
        )
        out_r[b] = outer + c0


_final = pl.pallas_call(
    _final_body,
    grid=(GRID_N,),
    in_specs=[
        pl.BlockSpec((NC, B, BLK), lambda i: (0, 0, i)),
        pl.BlockSpec((B, BLK), lambda i: (0, i)),
        pl.BlockSpec((1, BLK), lambda i: (0, i)),
        pl.BlockSpec((1, HID), lambda i: (0, 0)),
        pl.BlockSpec((HID, HOR), lambda i: (0, 0)),
        pl.BlockSpec((1, HID), lambda i: (0, 0)),
        pl.BlockSpec((1, HOR), lambda i: (0, 0)),
    ],
    out_specs=pl.BlockSpec((B, BLK, HOR), lambda i: (0, i, 0)),
    out_shape=jax.ShapeDtypeStruct((B, N, HOR), jnp.float32),
)


def kernel(x, edge_index, W_gcn, b_gcn, W_proj, b_proj):
    ei = edge_index.astype(jnp.int32)
    pad = jnp.full((E_ROWS * LGRP - E,), N_PAD - 1, jnp.int32)
    rows2d = jnp.concatenate([ei[0], pad]).reshape(E_ROWS, LGRP)
    cols2d = jnp.concatenate([ei[1], pad]).reshape(E_ROWS, LGRP)
    zeros1 = jnp.zeros((N_PAD,), jnp.float32)
    x3 = x[..., 0]

    degs_p, degc_p = _degrees(rows2d, cols2d, zeros1)
    gt4, dinv1 = _coef(x3, degs_p, degc_p)
    y_p = _edge_pass(rows2d, cols2d, gt4, zeros1)
    return _final(
        y_p, gt4, dinv1,
        W_gcn, W_proj,
        b_gcn.reshape(1, HID), b_proj.reshape(1, HOR),
    )
